# Initial kernel scaffold; baseline (speedup 1.0000x reference)
import jax
import jax.numpy as jnp
from jax.experimental import pallas as pl


def kernel(y_hat, keys, gt_idx):
    raise NotImplementedError("write your pallas kernel here")


# speedup vs baseline: 42.3683x; 42.3683x over previous
import jax
import jax.numpy as jnp
from jax.experimental import pallas as pl
from jax.experimental.pallas import tpu as pltpu

B, K, D, TOPK = 1024, 100000, 1024, 100
BK = 1024
NBLK = (K + BK - 1) // BK


def _qdiv_body(y_ref, nq_ref, qn_ref):
    qn_ref[...] = y_ref[...] / nq_ref[...]


def _diag_body(qn_ref, g_ref, ng_ref, sgt_ref):
    gn = g_ref[...] / ng_ref[...]
    s = jax.lax.dot_general(qn_ref[...], gn, (((1,), (1,)), ((), ())),
                            preferred_element_type=jnp.float32)
    mask = (jax.lax.broadcasted_iota(jnp.int32, (B, B), 0)
            == jax.lax.broadcasted_iota(jnp.int32, (B, B), 1))
    sgt_ref[...] = jnp.sum(jnp.where(mask, s, 0.0), axis=1, keepdims=True)


def _main_body(qn_ref, kb_ref, nk_ref, gt_ref, sgt_ref, out_ref, cnt_ref):
    i = pl.program_id(0)

    @pl.when(i == 0)
    def _():
        cnt_ref[...] = jnp.zeros_like(cnt_ref)

    kn = kb_ref[...] / nk_ref[...]
    s = jax.lax.dot_general(qn_ref[...], kn, (((1,), (1,)), ((), ())),
                            preferred_element_type=jnp.float32)
    sgt = sgt_ref[...]
    col = i * BK + jax.lax.broadcasted_iota(jnp.int32, (B, BK), 1)
    above = (s > sgt) & (col < K)
    tied = (s == sgt) & (col < gt_ref[...])
    cnt_ref[...] += jnp.sum((above | tied).astype(jnp.float32), axis=1,
                            keepdims=True)

    @pl.when(i == NBLK - 1)
    def _():
        rank = cnt_ref[...] + 1.0
        rr = jnp.where(rank <= TOPK, 1.0 / rank, 0.0)
        out_ref[0, 0] = jnp.sum(rr) / B


def kernel(y_hat, keys, gt_idx):
    gt = gt_idx.astype(jnp.int32)
    gt2d = gt.reshape(B, 1)
    nq = jnp.linalg.norm(y_hat, axis=-1, keepdims=True) + 1e-12
    nk = jnp.linalg.norm(keys, axis=-1, keepdims=True) + 1e-12
    g = jnp.take(keys, gt, axis=0)
    ng = jnp.take(nk, gt, axis=0)

    qn = pl.pallas_call(
        _qdiv_body,
        out_shape=jax.ShapeDtypeStruct((B, D), jnp.float32),
    )(y_hat, nq)

    sgt = pl.pallas_call(
        _diag_body,
        out_shape=jax.ShapeDtypeStruct((B, 1), jnp.float32),
    )(qn, g, ng)

    out = pl.pallas_call(
        _main_body,
        grid=(NBLK,),
        in_specs=[
            pl.BlockSpec((B, D), lambda i: (0, 0)),
            pl.BlockSpec((BK, D), lambda i: (i, 0)),
            pl.BlockSpec((BK, 1), lambda i: (i, 0)),
            pl.BlockSpec((B, 1), lambda i: (0, 0)),
            pl.BlockSpec((B, 1), lambda i: (0, 0)),
        ],
        out_specs=pl.BlockSpec(memory_space=pltpu.SMEM),
        out_shape=jax.ShapeDtypeStruct((1, 1), jnp.float32),
        scratch_shapes=[pltpu.VMEM((B, 1), jnp.float32)],
    )(qn, keys, nk, gt2d, sgt)

    return out[0, 0]
